# baseline (speedup 1.0000x reference)
import functools

import jax
import jax.numpy as jnp
from jax.experimental import pallas as pl
from jax.experimental.pallas import tpu as pltpu

_C = 16
_UNK = 1


def _fused_kernel(x_ref, wih_ref, whh_ref, b_ref, mask_ref,
                  sims_ref, top_ref, hall_ref, *, E, S, GB):
    C = _C
    n = GB * (C + 1)
    nsrc = GB * C

    wih = wih_ref[...]
    whh = whh_ref[...]
    bias = b_ref[...]

    h = jnp.zeros((n, E), jnp.float32)
    c = jnp.zeros((n, E), jnp.float32)

    for t in range(S):
        x_t = x_ref[:, t * E:(t + 1) * E]
        gates = (jnp.dot(x_t, wih, preferred_element_type=jnp.float32)
                 + jnp.dot(h, whh, preferred_element_type=jnp.float32)
                 + bias)
        sig_if = jax.nn.sigmoid(gates[:, :2 * E])
        g_g = jnp.tanh(gates[:, 2 * E:3 * E])
        o_g = jax.nn.sigmoid(gates[:, 3 * E:])
        i_g = sig_if[:, :E]
        f_g = sig_if[:, E:]
        c = f_g * c + i_g * g_g
        h = o_g * jnp.tanh(c)
        hall_ref[:, t * E:(t + 1) * E] = h.astype(hall_ref.dtype)

    qmask = mask_ref[...].astype(jnp.float32)
    qm = hall_ref[nsrc:n, :].astype(jnp.float32) * qmask

    s = jnp.zeros((GB, C), jnp.float32)
    KCH = min(2048, S * E)
    for j in range(0, S * E, KCH):
        src_j = hall_ref[:nsrc, j:j + KCH].astype(jnp.float32)
        src_j = src_j.reshape(GB, C, KCH)
        s = s + jnp.sum(src_j * qm[:, None, j:j + KCH], axis=-1)

    m = jnp.max(s, axis=-1, keepdims=True)
    e = jnp.exp(s - m)
    sims_ref[...] = e / jnp.sum(e, axis=-1, keepdims=True)
    idx = jax.lax.broadcasted_iota(jnp.int32, s.shape, 1)
    top_ref[...] = jnp.min(jnp.where(s == m, idx, jnp.int32(C)),
                           axis=-1, keepdims=True)


def kernel(sources, queries, embedding, w_ih, w_hh, b):
    C = _C
    B, S = queries.shape
    V, E = embedding.shape
    GB = 16 if B % 16 == 0 else (8 if B % 8 == 0 else B)

    src_ids = jnp.where(sources >= V, _UNK, sources)
    q_ids = jnp.where(queries >= V, _UNK, queries)

    emb_src = jnp.take(embedding, src_ids, axis=0)
    emb_q = jnp.take(embedding, q_ids, axis=0)

    N = B * C + B
    x = jnp.concatenate(
        [emb_src.reshape(B // GB, GB * C, S * E),
         emb_q.reshape(B // GB, GB, S * E)], axis=1).reshape(N, S * E)

    q_len = jnp.sum((queries > 0).astype(jnp.int32), axis=1)
    mask = jnp.arange(S)[None, :] < q_len[:, None]
    mask_flat = (jnp.broadcast_to(mask[:, :, None], (B, S, E))
                 .reshape(B, S * E).astype(jnp.bfloat16))

    grid = (B // GB,)
    n_rows = GB * (C + 1)
    body = functools.partial(_fused_kernel, E=E, S=S, GB=GB)
    sims, top = pl.pallas_call(
        body,
        out_shape=(jax.ShapeDtypeStruct((B, C), jnp.float32),
                   jax.ShapeDtypeStruct((B, 1), jnp.int32)),
        grid=grid,
        in_specs=[
            pl.BlockSpec((n_rows, S * E), lambda g: (g, 0)),
            pl.BlockSpec((E, 4 * E), lambda g: (0, 0)),
            pl.BlockSpec((E, 4 * E), lambda g: (0, 0)),
            pl.BlockSpec((1, 4 * E), lambda g: (0, 0)),
            pl.BlockSpec((GB, S * E), lambda g: (g, 0)),
        ],
        out_specs=(pl.BlockSpec((GB, C), lambda g: (g, 0)),
                   pl.BlockSpec((GB, 1), lambda g: (g, 0))),
        scratch_shapes=[pltpu.VMEM((n_rows, S * E), jnp.bfloat16)],
        compiler_params=pltpu.CompilerParams(
            dimension_semantics=("parallel",),
            vmem_limit_bytes=100 * 1024 * 1024,
        ),
    )(x, w_ih, w_hh, b, mask_flat)

    offsets = jnp.arange(B, dtype=jnp.int32) * C
    selected = jnp.take(sources, offsets + top[:, 0], axis=0)
    return selected, sims


# speedup vs baseline: 1.5433x; 1.5433x over previous
import functools

import jax
import jax.numpy as jnp
from jax.experimental import pallas as pl
from jax.experimental.pallas import tpu as pltpu

_C = 16
_UNK = 1


def _fused_kernel(ids_ref, table_ref, wih_ref, whh_ref, b_ref, mask_ref,
                  sims_ref, top_ref, hall_ref, xa_ref, xb_ref, *, E, S, GB):
    C = _C
    n = GB * (C + 1)
    nsrc = GB * C
    g = pl.program_id(0)
    row0 = g * n

    t_shift = S.bit_length() - 1

    def gather_iter(i, carry):
        rt = i >> t_shift
        t = i & (S - 1)
        base = rt * 8
        rows_a, rows_b = [], []
        for j in range(8):
            flat = (row0 + base + j) * S + t
            idx2 = pl.multiple_of(ids_ref[flat >> 7, flat & 127], 2)
            slab = table_ref[pl.ds(idx2, 2), :]
            rows_a.append(slab[0:1, :])
            rows_b.append(slab[1:2, :])
        xa_ref[t, pl.ds(base, 8), :] = jnp.concatenate(rows_a, axis=0)
        xb_ref[t, pl.ds(base, 8), :] = jnp.concatenate(rows_b, axis=0)
        return carry

    jax.lax.fori_loop(0, (n // 8) * S, gather_iter, 0)

    wih = wih_ref[...]
    whh = whh_ref[...]
    bias = b_ref[...]

    h = jnp.zeros((n, E), jnp.float32)
    c = jnp.zeros((n, E), jnp.float32)

    for t in range(S):
        x_t = jnp.concatenate([xa_ref[t], xb_ref[t]], axis=-1)
        gates = (jnp.dot(x_t, wih, preferred_element_type=jnp.float32)
                 + jnp.dot(h, whh, preferred_element_type=jnp.float32)
                 + bias)
        sig_if = jax.nn.sigmoid(gates[:, :2 * E])
        g_g = jnp.tanh(gates[:, 2 * E:3 * E])
        o_g = jax.nn.sigmoid(gates[:, 3 * E:])
        i_g = sig_if[:, :E]
        f_g = sig_if[:, E:]
        c = f_g * c + i_g * g_g
        h = o_g * jnp.tanh(c)
        hall_ref[:, t * E:(t + 1) * E] = h.astype(hall_ref.dtype)

    qmask = mask_ref[...].astype(jnp.float32)
    qm = hall_ref[nsrc:n, :].astype(jnp.float32) * qmask

    s = jnp.zeros((GB, C), jnp.float32)
    KCH = min(2048, S * E)
    for j in range(0, S * E, KCH):
        src_j = hall_ref[:nsrc, j:j + KCH].astype(jnp.float32)
        src_j = src_j.reshape(GB, C, KCH)
        s = s + jnp.sum(src_j * qm[:, None, j:j + KCH], axis=-1)

    m = jnp.max(s, axis=-1, keepdims=True)
    e = jnp.exp(s - m)
    sims_ref[...] = e / jnp.sum(e, axis=-1, keepdims=True)
    idx = jax.lax.broadcasted_iota(jnp.int32, s.shape, 1)
    top_ref[...] = jnp.min(jnp.where(s == m, idx, jnp.int32(C)),
                           axis=-1, keepdims=True)


def kernel(sources, queries, embedding, w_ih, w_hh, b):
    C = _C
    B, S = queries.shape
    V, E = embedding.shape
    GB = 16 if B % 16 == 0 else (8 if B % 8 == 0 else B)

    src_ids = jnp.where(sources >= V, _UNK, sources)
    q_ids = jnp.where(queries >= V, _UNK, queries)

    sid3 = src_ids.reshape(B // GB, GB * C, S)
    qid3 = q_ids.reshape(B // GB, GB, S)
    ids2 = jnp.concatenate([sid3, qid3], axis=1).reshape(-1, 128) * 2

    table2 = embedding.reshape(2 * V, E // 2)

    q_len = jnp.sum((queries > 0).astype(jnp.int32), axis=1)
    mask = jnp.arange(S)[None, :] < q_len[:, None]
    mask_flat = (jnp.broadcast_to(mask[:, :, None], (B, S, E))
                 .reshape(B, S * E).astype(jnp.bfloat16))

    grid = (B // GB,)
    n_rows = GB * (C + 1)
    body = functools.partial(_fused_kernel, E=E, S=S, GB=GB)
    sims, top = pl.pallas_call(
        body,
        out_shape=(jax.ShapeDtypeStruct((B, C), jnp.float32),
                   jax.ShapeDtypeStruct((B, 1), jnp.int32)),
        grid_spec=pltpu.PrefetchScalarGridSpec(
            num_scalar_prefetch=1,
            grid=grid,
            in_specs=[
                pl.BlockSpec((2 * V, E // 2), lambda g, ids: (0, 0)),
                pl.BlockSpec((E, 4 * E), lambda g, ids: (0, 0)),
                pl.BlockSpec((E, 4 * E), lambda g, ids: (0, 0)),
                pl.BlockSpec((1, 4 * E), lambda g, ids: (0, 0)),
                pl.BlockSpec((GB, S * E), lambda g, ids: (g, 0)),
            ],
            out_specs=(pl.BlockSpec((GB, C), lambda g, ids: (g, 0)),
                       pl.BlockSpec((GB, 1), lambda g, ids: (g, 0))),
            scratch_shapes=[
                pltpu.VMEM((n_rows, S * E), jnp.bfloat16),
                pltpu.VMEM((S, n_rows, E // 2), jnp.float32),
                pltpu.VMEM((S, n_rows, E // 2), jnp.float32),
            ],
        ),
        compiler_params=pltpu.CompilerParams(
            dimension_semantics=("parallel",),
            vmem_limit_bytes=100 * 1024 * 1024,
        ),
    )(ids2, table2, w_ih, w_hh, b, mask_flat)

    offsets = jnp.arange(B, dtype=jnp.int32) * C
    selected = jnp.take(sources, offsets + top[:, 0], axis=0)
    return selected, sims
